# Initial kernel scaffold; baseline (speedup 1.0000x reference)
import jax
import jax.numpy as jnp
from jax.experimental import pallas as pl


def kernel(indices, edge_index, root_idx, emb, W_conv, b_conv, W_func, b_func):
    raise NotImplementedError("write your pallas kernel here")


# speedup vs baseline: 13.7777x; 13.7777x over previous
import jax
import jax.numpy as jnp
from jax import lax
from jax.experimental import pallas as pl
from jax.experimental.pallas import tpu as pltpu
from jax.experimental.pallas import tpu_sc as plsc

N = 10000
E = 320000
DIM = 128
VOCAB = 500
NB = 16

NPAD = 10240
EROWS = 2560
EPAD = EROWS * 128
R_PER_TILE = 160
E_PER_TILE = R_PER_TILE * 128
CH = 2048
NBLK = R_PER_TILE // 16

VP = 512
RP = 2500
DSZ = 8192 * 160
DDUMP = RP * VP
CSZ = 327680
CDUMP = N * 32

_MESH = dict(core_axis_name="c", subcore_axis_name="s")


def _iota16():
    return lax.broadcasted_iota(jnp.int32, (16,), 0)


def _deg_body(dst_hbm, deg_hbm, deg_sp, dstb, idxb, valb, zb, sem):
    c = lax.axis_index("c")
    s = lax.axis_index("s")

    def zf(i, _):
        zb[pl.ds(i * 16, 16)] = jnp.zeros((16,), jnp.float32)
        return 0
    lax.fori_loop(0, 40, zf, 0)
    pltpu.sync_copy(zb, deg_sp.at[pl.ds(s * 640, 640)])
    plsc.subcore_barrier()

    def blk_body(blk, _):
        rbase = s * R_PER_TILE + blk * 16
        pltpu.sync_copy(dst_hbm.at[pl.ds(rbase, 16)], dstb)
        for r in range(16):
            def qf(q, _):
                oi = (rbase + r) * 128 + q * 16 + _iota16()
                valid = oi < E
                d16 = dstb[r, pl.ds(q * 16, 16)]
                idxb[r, pl.ds(q * 16, 16)] = jnp.where(
                    valid, d16, N + (oi & 127))
                valb[r, pl.ds(q * 16, 16)] = jnp.where(valid, 1.0, 0.0)
                return 0
            lax.fori_loop(0, 8, qf, 0)
        copies = [
            pltpu.async_copy(valb.at[r], deg_sp.at[idxb.at[r]], sem, add=True)
            for r in range(16)
        ]
        for h in copies:
            h.wait()
        return 0

    lax.fori_loop(0, NBLK, blk_body, 0)
    plsc.subcore_barrier()

    @pl.when(c == 0)
    def _():
        pltpu.sync_copy(deg_sp.at[pl.ds(s * 640, 640)],
                        deg_hbm.at[pl.ds(s * 640, 640)])


def _sc_deg(dst2d):
    return pl.kernel(
        _deg_body,
        out_type=jax.ShapeDtypeStruct((NPAD,), jnp.float32),
        mesh=plsc.VectorSubcoreMesh(**_MESH),
        scratch_types=[
            pltpu.VMEM_SHARED((NPAD,), jnp.float32),
            pltpu.VMEM((16, 128), jnp.int32),
            pltpu.VMEM((16, 128), jnp.int32),
            pltpu.VMEM((16, 128), jnp.float32),
            pltpu.VMEM((640,), jnp.float32),
            pltpu.SemaphoreType.DMA,
        ],
    )(dst2d)


def _norm_body(deg_ref, norm_ref):
    norm_ref[...] = lax.rsqrt(jnp.maximum(deg_ref[...], 1.0))


def _tc_norm(deg):
    return pl.pallas_call(
        _norm_body,
        out_shape=jax.ShapeDtypeStruct((NPAD // 128, 128), jnp.float32),
    )(deg.reshape(NPAD // 128, 128)).reshape(NPAD)


def _build_body(src_hbm, dst_hbm, ind_hbm, slot_hbm, norm_hbm,
                dflat_hbm, cflat_hbm,
                d_sp, c_sp, ind_sp, slot_sp, norm_sp,
                srcb, dstb, vvb, slb, nsb, ndb,
                idxD, valD, idxC, valC, zb, sem):
    c = lax.axis_index("c")
    s = lax.axis_index("s")

    def zf(i, _):
        zb[pl.ds(i * 16, 16)] = jnp.zeros((16,), jnp.float32)
        return 0
    lax.fori_loop(0, 128, zf, 0)

    pltpu.sync_copy(ind_hbm.at[pl.ds(s * 640, 640)],
                    ind_sp.at[pl.ds(s * 640, 640)])
    pltpu.sync_copy(slot_hbm.at[pl.ds(s * 640, 640)],
                    slot_sp.at[pl.ds(s * 640, 640)])
    pltpu.sync_copy(norm_hbm.at[pl.ds(s * 640, 640)],
                    norm_sp.at[pl.ds(s * 640, 640)])

    def pass_body(p, _):
        rowbase = c * (2 * RP) + p * RP

        def zd(k, _):
            pltpu.sync_copy(zb, d_sp.at[pl.ds((s * 40 + k) * 2048, 2048)])
            return 0
        lax.fori_loop(0, 40, zd, 0)

        @pl.when(p == 0)
        def _():
            def zc(k, _):
                pltpu.sync_copy(zb, c_sp.at[pl.ds((s * 10 + k) * 2048, 2048)])
                return 0
            lax.fori_loop(0, 10, zc, 0)

        plsc.subcore_barrier()

        def blk_body(blk, _):
            rbase = s * R_PER_TILE + blk * 16
            pltpu.sync_copy(src_hbm.at[pl.ds(rbase, 16)], srcb)
            pltpu.sync_copy(dst_hbm.at[pl.ds(rbase, 16)], dstb)

            gathers = []
            for r in range(16):
                gathers.append(pltpu.async_copy(
                    ind_sp.at[srcb.at[r]], vvb.at[r], sem))
                gathers.append(pltpu.async_copy(
                    norm_sp.at[srcb.at[r]], nsb.at[r], sem))
                gathers.append(pltpu.async_copy(
                    norm_sp.at[dstb.at[r]], ndb.at[r], sem))
                gathers.append(pltpu.async_copy(
                    slot_sp.at[dstb.at[r]], slb.at[r], sem))
            for h in gathers:
                h.wait()

            for r in range(16):
                def qf(q, _):
                    oi = (rbase + r) * 128 + q * 16 + _iota16()
                    valid = oi < E
                    dm = dstb[r, pl.ds(q * 16, 16)]
                    v16 = vvb[r, pl.ds(q * 16, 16)]
                    ns = nsb[r, pl.ds(q * 16, 16)]
                    nd = ndb[r, pl.ds(q * 16, 16)]
                    sl = slb[r, pl.ds(q * 16, 16)]
                    w = ns * nd
                    inr = valid & (dm >= rowbase) & (dm < rowbase + RP)
                    idxD[r, pl.ds(q * 16, 16)] = jnp.where(
                        inr, (dm - rowbase) * VP + v16,
                        DDUMP + (oi & 8191))
                    valD[r, pl.ds(q * 16, 16)] = jnp.where(inr, w, 0.0)
                    sm = srcb[r, pl.ds(q * 16, 16)]
                    idxC[r, pl.ds(q * 16, 16)] = jnp.where(
                        valid, sm * 32 + sl, CDUMP + (oi & 4095))
                    valC[r, pl.ds(q * 16, 16)] = jnp.where(valid, w, 0.0)
                    return 0
                lax.fori_loop(0, 8, qf, 0)

            copies = [
                pltpu.async_copy(valD.at[r], d_sp.at[idxD.at[r]], sem, add=True)
                for r in range(16)
            ]
            for h in copies:
                h.wait()

            @pl.when((c == 0) & (p == 0))
            def _():
                copies_c = [
                    pltpu.async_copy(valC.at[r], c_sp.at[idxC.at[r]], sem,
                                     add=True)
                    for r in range(16)
                ]
                for h in copies_c:
                    h.wait()
            return 0

        lax.fori_loop(0, NBLK, blk_body, 0)
        plsc.subcore_barrier()

        hbase = rowbase * VP
        pltpu.sync_copy(
            d_sp.at[pl.ds(s * 80000, 80000)],
            dflat_hbm.at[pl.ds(hbase + s * 80000, 80000)])

        @pl.when((c == 0) & (p == 0))
        def _():
            pltpu.sync_copy(
                c_sp.at[pl.ds(s * 20480, 20480)],
                cflat_hbm.at[pl.ds(s * 20480, 20480)])

        plsc.subcore_barrier()
        return 0

    lax.fori_loop(0, 2, pass_body, 0)


def _sc_build(src2d, dst2d, indp, slot, norm):
    return pl.kernel(
        _build_body,
        out_type=(
            jax.ShapeDtypeStruct((N * VP,), jnp.float32),
            jax.ShapeDtypeStruct((CSZ,), jnp.float32),
        ),
        mesh=plsc.VectorSubcoreMesh(**_MESH),
        scratch_types=[
            pltpu.VMEM_SHARED((DSZ,), jnp.float32),
            pltpu.VMEM_SHARED((CSZ,), jnp.float32),
            pltpu.VMEM_SHARED((NPAD,), jnp.int32),
            pltpu.VMEM_SHARED((NPAD,), jnp.int32),
            pltpu.VMEM_SHARED((NPAD,), jnp.float32),
            pltpu.VMEM((16, 128), jnp.int32),
            pltpu.VMEM((16, 128), jnp.int32),
            pltpu.VMEM((16, 128), jnp.int32),
            pltpu.VMEM((16, 128), jnp.int32),
            pltpu.VMEM((16, 128), jnp.float32),
            pltpu.VMEM((16, 128), jnp.float32),
            pltpu.VMEM((16, 128), jnp.int32),
            pltpu.VMEM((16, 128), jnp.float32),
            pltpu.VMEM((16, 128), jnp.int32),
            pltpu.VMEM((16, 128), jnp.float32),
            pltpu.VMEM((2048,), jnp.float32),
            pltpu.SemaphoreType.DMA,
        ],
    )(src2d, dst2d, indp, slot, norm)


def _tc_body(d_ref, c_ref, emb_ref, w_ref, b_ref, root_ref, g_ref,
             wf_ref, bf_ref, out_ref, m2_acc, r1_acc):
    i = pl.program_id(0)

    @pl.when(i == 0)
    def _():
        m2_acc[...] = jnp.zeros((32, DIM), jnp.float32)
        r1_acc[...] = jnp.zeros((16, DIM), jnp.float32)

    msg = jnp.dot(d_ref[...], emb_ref[...], preferred_element_type=jnp.float32)
    x1 = jnp.maximum(
        jnp.dot(msg, w_ref[...], preferred_element_type=jnp.float32)
        + b_ref[...], 0.0)
    m2_acc[...] += lax.dot_general(
        c_ref[...], x1, (((0,), (0,)), ((), ())),
        preferred_element_type=jnp.float32)
    rows = lax.broadcasted_iota(jnp.int32, (1000, 16), 0) + i * 1000
    oh = (rows == root_ref[...]).astype(jnp.float32)
    r1_acc[...] += lax.dot_general(
        oh, x1, (((0,), (0,)), ((), ())), preferred_element_type=jnp.float32)

    @pl.when(i == 9)
    def _():
        m2r = jnp.dot(g_ref[...], m2_acc[...],
                      preferred_element_type=jnp.float32)
        x2r = jnp.maximum(
            jnp.dot(m2r, w_ref[...], preferred_element_type=jnp.float32)
            + b_ref[...], 0.0)
        out_ref[...] = jnp.dot(
            r1_acc[...] + x2r, wf_ref[...],
            preferred_element_type=jnp.float32) + bf_ref[...]


def _tc_dense(Dn, C2, emb, W, b2, root2, G, Wf, bf2):
    return pl.pallas_call(
        _tc_body,
        grid=(10,),
        in_specs=[
            pl.BlockSpec((1000, VP), lambda i: (i, 0)),
            pl.BlockSpec((1000, 32), lambda i: (i, 0)),
            pl.BlockSpec((VP, DIM), lambda i: (0, 0)),
            pl.BlockSpec((DIM, DIM), lambda i: (0, 0)),
            pl.BlockSpec((1, DIM), lambda i: (0, 0)),
            pl.BlockSpec((1, 16), lambda i: (0, 0)),
            pl.BlockSpec((16, 32), lambda i: (0, 0)),
            pl.BlockSpec((DIM, 2), lambda i: (0, 0)),
            pl.BlockSpec((1, 2), lambda i: (0, 0)),
        ],
        out_specs=pl.BlockSpec((16, 2), lambda i: (0, 0)),
        out_shape=jax.ShapeDtypeStruct((16, 2), jnp.float32),
        scratch_shapes=[
            pltpu.VMEM((32, DIM), jnp.float32),
            pltpu.VMEM((16, DIM), jnp.float32),
        ],
    )(Dn, C2, emb, W, b2, root2, G, Wf, bf2)


def kernel(indices, edge_index, root_idx, emb, W_conv, b_conv, W_func, b_func):
    src = edge_index[0]
    dst = edge_index[1]
    pad = jnp.zeros((EPAD - E,), jnp.int32)
    src2d = jnp.concatenate([src, pad]).reshape(EROWS, 128)
    dst2d = jnp.concatenate([dst, pad]).reshape(EROWS, 128)

    slot = jnp.full((NPAD,), 16, jnp.int32).at[root_idx].set(
        jnp.arange(NB, dtype=jnp.int32))
    indp = jnp.concatenate([indices, jnp.zeros((NPAD - N,), jnp.int32)])
    slot_of_root = slot[root_idx]
    G = (slot_of_root[:, None]
         == jnp.arange(32, dtype=jnp.int32)[None, :]).astype(jnp.float32)

    norm = _tc_norm(_sc_deg(dst2d))
    Dflat, Cflat = _sc_build(src2d, dst2d, indp, slot, norm)

    Dn = Dflat.reshape(N, VP)
    C2 = Cflat[: N * 32].reshape(N, 32)
    embp = jnp.concatenate(
        [emb, jnp.zeros((VP - VOCAB, DIM), jnp.float32)], axis=0)
    out = _tc_dense(
        Dn, C2, embp, W_conv, b_conv.reshape(1, DIM),
        root_idx.reshape(1, 16), G, W_func, b_func.reshape(1, 2))
    return out


# speedup vs baseline: 15.0523x; 1.0925x over previous
import jax
import jax.numpy as jnp
from jax import lax
from jax.experimental import pallas as pl
from jax.experimental.pallas import tpu as pltpu
from jax.experimental.pallas import tpu_sc as plsc

N = 10000
E = 320000
DIM = 128
VOCAB = 500
NB = 16

NPAD = 10240
EROWS = 2560
EPAD = EROWS * 128
R_PER_TILE = 160
E_PER_TILE = R_PER_TILE * 128
CH = 2048
NBLK = R_PER_TILE // 16

VP = 512
RP = 2500
DSZ = 8192 * 160
DDUMP = RP * VP
CSZ = 327680
CDUMP = N * 32

_MESH = dict(core_axis_name="c", subcore_axis_name="s")


def _iota16():
    return lax.broadcasted_iota(jnp.int32, (16,), 0)


def _deg_body(dst_hbm, deg_hbm, deg_sp, dstb, idxb, valb, zb, sem):
    c = lax.axis_index("c")
    s = lax.axis_index("s")

    def zf(i, _):
        zb[pl.ds(i * 16, 16)] = jnp.zeros((16,), jnp.float32)
        return 0
    lax.fori_loop(0, 40, zf, 0)
    pltpu.sync_copy(zb, deg_sp.at[pl.ds(s * 640, 640)])
    plsc.subcore_barrier()

    def blk_body(blk, _):
        rbase = s * R_PER_TILE + blk * 16
        pltpu.sync_copy(dst_hbm.at[pl.ds(rbase, 16)], dstb)
        for r in range(16):
            def qf(q, _):
                oi = (rbase + r) * 128 + q * 16 + _iota16()
                valid = oi < E
                d16 = dstb[r, pl.ds(q * 16, 16)]
                idxb[r, pl.ds(q * 16, 16)] = jnp.where(
                    valid, d16, N + (oi & 127))
                valb[r, pl.ds(q * 16, 16)] = jnp.where(valid, 1.0, 0.0)
                return 0
            lax.fori_loop(0, 8, qf, 0)
        copies = [
            pltpu.async_copy(valb.at[r], deg_sp.at[idxb.at[r]], sem, add=True)
            for r in range(16)
        ]
        for h in copies:
            h.wait()
        return 0

    lax.fori_loop(0, NBLK, blk_body, 0)
    plsc.subcore_barrier()

    @pl.when(c == 0)
    def _():
        pltpu.sync_copy(deg_sp.at[pl.ds(s * 640, 640)],
                        deg_hbm.at[pl.ds(s * 640, 640)])


def _sc_deg(dst2d):
    return pl.kernel(
        _deg_body,
        out_type=jax.ShapeDtypeStruct((NPAD,), jnp.float32),
        mesh=plsc.VectorSubcoreMesh(**_MESH),
        scratch_types=[
            pltpu.VMEM_SHARED((NPAD,), jnp.float32),
            pltpu.VMEM((16, 128), jnp.int32),
            pltpu.VMEM((16, 128), jnp.int32),
            pltpu.VMEM((16, 128), jnp.float32),
            pltpu.VMEM((640,), jnp.float32),
            pltpu.SemaphoreType.DMA,
        ],
    )(dst2d)


def _norm_body(deg_ref, norm_ref):
    norm_ref[...] = lax.rsqrt(jnp.maximum(deg_ref[...], 1.0))


def _tc_norm(deg):
    return pl.pallas_call(
        _norm_body,
        out_shape=jax.ShapeDtypeStruct((NPAD // 128, 128), jnp.float32),
    )(deg.reshape(NPAD // 128, 128)).reshape(NPAD)


def _build_body(src_hbm, dst_hbm, ind_hbm, slot_hbm, norm_hbm,
                dflat_hbm, cflat_hbm,
                d_sp, c_sp, ind_sp, slot_sp, norm_sp,
                srcb, dstb, vvb, slb, nsb,
                idxD, valD, idxC, valC, zb, sem):
    c = lax.axis_index("c")
    s = lax.axis_index("s")

    def zf(i, _):
        zb[pl.ds(i * 16, 16)] = jnp.zeros((16,), jnp.float32)
        return 0
    lax.fori_loop(0, 128, zf, 0)

    pltpu.sync_copy(ind_hbm.at[pl.ds(s * 640, 640)],
                    ind_sp.at[pl.ds(s * 640, 640)])
    pltpu.sync_copy(norm_hbm.at[pl.ds(s * 640, 640)],
                    norm_sp.at[pl.ds(s * 640, 640)])

    @pl.when(c == 0)
    def _():
        pltpu.sync_copy(slot_hbm.at[pl.ds(s * 640, 640)],
                        slot_sp.at[pl.ds(s * 640, 640)])

    def pass_body(p, _):
        rowbase = c * (2 * RP) + p * RP
        build_c = (c == 0) & (p == 0)

        def zd(k, _):
            pltpu.sync_copy(zb, d_sp.at[pl.ds((s * 40 + k) * 2048, 2048)])
            return 0
        lax.fori_loop(0, 40, zd, 0)

        @pl.when(build_c)
        def _():
            def zc(k, _):
                pltpu.sync_copy(zb, c_sp.at[pl.ds((s * 10 + k) * 2048, 2048)])
                return 0
            lax.fori_loop(0, 10, zc, 0)

        plsc.subcore_barrier()

        def blk_body(blk, _):
            rbase = s * R_PER_TILE + blk * 16
            pltpu.sync_copy(src_hbm.at[pl.ds(rbase, 16)], srcb)
            pltpu.sync_copy(dst_hbm.at[pl.ds(rbase, 16)], dstb)

            gathers = []
            for r in range(16):
                gathers.append(pltpu.async_copy(
                    ind_sp.at[srcb.at[r]], vvb.at[r], sem))
                gathers.append(pltpu.async_copy(
                    norm_sp.at[srcb.at[r]], nsb.at[r], sem))
            for h in gathers:
                h.wait()

            for r in range(16):
                def qf(q, _):
                    oi = (rbase + r) * 128 + q * 16 + _iota16()
                    valid = oi < E
                    dm = dstb[r, pl.ds(q * 16, 16)]
                    v16 = vvb[r, pl.ds(q * 16, 16)]
                    ns = nsb[r, pl.ds(q * 16, 16)]
                    inr = valid & (dm >= rowbase) & (dm < rowbase + RP)
                    idxD[r, pl.ds(q * 16, 16)] = jnp.where(
                        inr, (dm - rowbase) * VP + v16,
                        DDUMP + (oi & 8191))
                    valD[r, pl.ds(q * 16, 16)] = jnp.where(inr, ns, 0.0)
                    return 0
                lax.fori_loop(0, 8, qf, 0)

            copies = [
                pltpu.async_copy(valD.at[r], d_sp.at[idxD.at[r]], sem, add=True)
                for r in range(16)
            ]
            for h in copies:
                h.wait()

            @pl.when(build_c)
            def _():
                g2 = [
                    pltpu.async_copy(slot_sp.at[dstb.at[r]], slb.at[r], sem)
                    for r in range(16)
                ]
                for h in g2:
                    h.wait()
                for r in range(16):
                    def qc(q, _):
                        oi = (rbase + r) * 128 + q * 16 + _iota16()
                        valid = oi < E
                        ns = nsb[r, pl.ds(q * 16, 16)]
                        sl = slb[r, pl.ds(q * 16, 16)]
                        sm = srcb[r, pl.ds(q * 16, 16)]
                        idxC[r, pl.ds(q * 16, 16)] = jnp.where(
                            valid, sm * 32 + sl, CDUMP + (oi & 4095))
                        valC[r, pl.ds(q * 16, 16)] = jnp.where(valid, ns, 0.0)
                        return 0
                    lax.fori_loop(0, 8, qc, 0)
                copies_c = [
                    pltpu.async_copy(valC.at[r], c_sp.at[idxC.at[r]], sem,
                                     add=True)
                    for r in range(16)
                ]
                for h in copies_c:
                    h.wait()
            return 0

        lax.fori_loop(0, NBLK, blk_body, 0)
        plsc.subcore_barrier()

        hbase = rowbase * VP
        pltpu.sync_copy(
            d_sp.at[pl.ds(s * 80000, 80000)],
            dflat_hbm.at[pl.ds(hbase + s * 80000, 80000)])

        @pl.when((c == 0) & (p == 0))
        def _():
            pltpu.sync_copy(
                c_sp.at[pl.ds(s * 20480, 20480)],
                cflat_hbm.at[pl.ds(s * 20480, 20480)])

        plsc.subcore_barrier()
        return 0

    lax.fori_loop(0, 2, pass_body, 0)


def _sc_build(src2d, dst2d, indp, slot, norm):
    return pl.kernel(
        _build_body,
        out_type=(
            jax.ShapeDtypeStruct((N * VP,), jnp.float32),
            jax.ShapeDtypeStruct((CSZ,), jnp.float32),
        ),
        mesh=plsc.VectorSubcoreMesh(**_MESH),
        scratch_types=[
            pltpu.VMEM_SHARED((DSZ,), jnp.float32),
            pltpu.VMEM_SHARED((CSZ,), jnp.float32),
            pltpu.VMEM_SHARED((NPAD,), jnp.int32),
            pltpu.VMEM_SHARED((NPAD,), jnp.int32),
            pltpu.VMEM_SHARED((NPAD,), jnp.float32),
            pltpu.VMEM((16, 128), jnp.int32),
            pltpu.VMEM((16, 128), jnp.int32),
            pltpu.VMEM((16, 128), jnp.int32),
            pltpu.VMEM((16, 128), jnp.int32),
            pltpu.VMEM((16, 128), jnp.float32),
            pltpu.VMEM((16, 128), jnp.int32),
            pltpu.VMEM((16, 128), jnp.float32),
            pltpu.VMEM((16, 128), jnp.int32),
            pltpu.VMEM((16, 128), jnp.float32),
            pltpu.VMEM((2048,), jnp.float32),
            pltpu.SemaphoreType.DMA,
        ],
    )(src2d, dst2d, indp, slot, norm)


def _tc_body(d_ref, c_ref, n_ref, emb_ref, w_ref, b_ref, root_ref, g_ref,
             wf_ref, bf_ref, out_ref, m2_acc, r1_acc):
    i = pl.program_id(0)

    @pl.when(i == 0)
    def _():
        m2_acc[...] = jnp.zeros((32, DIM), jnp.float32)
        r1_acc[...] = jnp.zeros((16, DIM), jnp.float32)

    msg = jnp.dot(d_ref[...], emb_ref[...],
                  preferred_element_type=jnp.float32) * n_ref[...]
    x1 = jnp.maximum(
        jnp.dot(msg, w_ref[...], preferred_element_type=jnp.float32)
        + b_ref[...], 0.0)
    m2_acc[...] += lax.dot_general(
        c_ref[...], x1, (((0,), (0,)), ((), ())),
        preferred_element_type=jnp.float32)
    rows = lax.broadcasted_iota(jnp.int32, (1000, 16), 0) + i * 1000
    oh = (rows == root_ref[...]).astype(jnp.float32)
    r1_acc[...] += lax.dot_general(
        oh, x1, (((0,), (0,)), ((), ())), preferred_element_type=jnp.float32)

    @pl.when(i == 9)
    def _():
        m2r = jnp.dot(g_ref[...], m2_acc[...],
                      preferred_element_type=jnp.float32)
        x2r = jnp.maximum(
            jnp.dot(m2r, w_ref[...], preferred_element_type=jnp.float32)
            + b_ref[...], 0.0)
        out_ref[...] = jnp.dot(
            r1_acc[...] + x2r, wf_ref[...],
            preferred_element_type=jnp.float32) + bf_ref[...]


def _tc_dense(Dn, C2, norm2d, emb, W, b2, root2, G, Wf, bf2):
    return pl.pallas_call(
        _tc_body,
        grid=(10,),
        in_specs=[
            pl.BlockSpec((1000, VP), lambda i: (i, 0)),
            pl.BlockSpec((1000, 32), lambda i: (i, 0)),
            pl.BlockSpec((1000, 1), lambda i: (i, 0)),
            pl.BlockSpec((VP, DIM), lambda i: (0, 0)),
            pl.BlockSpec((DIM, DIM), lambda i: (0, 0)),
            pl.BlockSpec((1, DIM), lambda i: (0, 0)),
            pl.BlockSpec((1, 16), lambda i: (0, 0)),
            pl.BlockSpec((16, 32), lambda i: (0, 0)),
            pl.BlockSpec((DIM, 2), lambda i: (0, 0)),
            pl.BlockSpec((1, 2), lambda i: (0, 0)),
        ],
        out_specs=pl.BlockSpec((16, 2), lambda i: (0, 0)),
        out_shape=jax.ShapeDtypeStruct((16, 2), jnp.float32),
        scratch_shapes=[
            pltpu.VMEM((32, DIM), jnp.float32),
            pltpu.VMEM((16, DIM), jnp.float32),
        ],
    )(Dn, C2, norm2d, emb, W, b2, root2, G, Wf, bf2)


def kernel(indices, edge_index, root_idx, emb, W_conv, b_conv, W_func, b_func):
    src = edge_index[0]
    dst = edge_index[1]
    pad = jnp.zeros((EPAD - E,), jnp.int32)
    src2d = jnp.concatenate([src, pad]).reshape(EROWS, 128)
    dst2d = jnp.concatenate([dst, pad]).reshape(EROWS, 128)

    slot = jnp.full((NPAD,), 16, jnp.int32).at[root_idx].set(
        jnp.arange(NB, dtype=jnp.int32))
    indp = jnp.concatenate([indices, jnp.zeros((NPAD - N,), jnp.int32)])
    slot_of_root = slot[root_idx]
    Goh = (slot_of_root[:, None]
           == jnp.arange(32, dtype=jnp.int32)[None, :]).astype(jnp.float32)

    norm = _tc_norm(_sc_deg(dst2d))
    G = Goh * norm[root_idx][:, None]
    Dflat, Cflat = _sc_build(src2d, dst2d, indp, slot, norm)

    Dn = Dflat.reshape(N, VP)
    C2 = Cflat[: N * 32].reshape(N, 32)
    norm2d = norm[:N].reshape(N, 1)
    embp = jnp.concatenate(
        [emb, jnp.zeros((VP - VOCAB, DIM), jnp.float32)], axis=0)
    out = _tc_dense(
        Dn, C2, norm2d, embp, W_conv, b_conv.reshape(1, DIM),
        root_idx.reshape(1, 16), G, W_func, b_func.reshape(1, 2))
    return out
